# baseline (speedup 1.0000x reference)
import functools

import jax
import jax.numpy as jnp
from jax import lax
from jax.experimental import pallas as pl
from jax.experimental.pallas import tpu as pltpu
from jax.experimental.pallas import tpu_sc as plsc

L = 16


@functools.cache
def _make_gather(B: int, H: int, V: int, D: int):
    info = plsc.get_sparse_core_info()
    nc, ns = info.num_cores, info.num_subcores
    nw = nc * ns
    nb = B // nw
    nbt = nb // 128
    dt, di = D // 8, 8
    assert nb * nw == B and nbt * 128 == nb
    mesh = plsc.VectorSubcoreMesh(core_axis_name="c", subcore_axis_name="s")

    @functools.partial(
        pl.kernel,
        mesh=mesh,
        out_type=jax.ShapeDtypeStruct((H, dt, B // 128, di, 128), jnp.float32),
        compiler_params=pltpu.CompilerParams(
            use_tc_tiling_on_sc=False, needs_layout_passes=False,
            disable_bounds_checks=True),
        scratch_types=[
            pltpu.VMEM((V * D * 8,), jnp.float32),
            pltpu.VMEM((nb * H,), jnp.int32),
            pltpu.VMEM((2, dt, nbt, di, 128), jnp.float32),
            pltpu.SemaphoreType.DMA((2,)),
        ],
    )
    def gather_kernel(tab8_hbm, idx_hbm, out_hbm, tab8_v, idx_v, stg_v, ssem):
        cid = lax.axis_index("c")
        sid = lax.axis_index("s")
        wid = sid * nc + cid

        pltpu.sync_copy(tab8_hbm, tab8_v)
        pltpu.sync_copy(idx_hbm.at[pl.ds(wid * nb * H, nb * H)], idx_v)

        iota = lax.iota(jnp.int32, L)
        iota8 = lax.rem(iota, jnp.full((L,), 8, jnp.int32))
        iota_h = iota * H

        def store_dst(h):
            return out_hbm.at[h, :, pl.ds(wid * nbt, nbt)]

        def start_store(p, h):
            pltpu.async_copy(stg_v.at[p], store_dst(h), ssem.at[p])

        def wait_store(p, h):
            pltpu.make_async_copy(stg_v.at[p], store_dst(h),
                                  ssem.at[p]).wait()

        def produce(p, h):
            @plsc.parallel_loop(0, nbt * (128 // L), 1, unroll=8)
            def _(c):
                btl = lax.shift_right_logical(c, 3)
                g = lax.bitwise_and(c, 7)
                t0 = btl * 128 + g * L
                idxvals = plsc.load_gather(idx_v, [iota_h + (t0 * H + h)])
                base = idxvals * (D * 8) + iota8
                for d in range(D):
                    vals = plsc.load_gather(tab8_v, [base + d * 8])
                    stg_v[p, d // di, btl, d % di, pl.ds(g * L, L)] = vals

        def block(j, carry):
            for p in range(2):
                h = 2 * j + p

                @pl.when(j > 0)
                def _():
                    wait_store(p, h - 2)

                produce(p, h)
                start_store(p, h)
            return carry

        lax.fori_loop(0, H // 2, block, 0)
        wait_store(0, H - 2)
        wait_store(1, H - 1)

    return gather_kernel


def kernel(inputs, table):
    batch, hist = inputs.shape
    rows, dim = table.shape
    flat_idx = inputs.reshape(batch * hist)
    table_8 = jnp.tile(table.reshape(rows * dim, 1), (1, 8)).reshape(-1)
    out5 = _make_gather(batch, hist, rows, dim)(table_8, flat_idx)
    out = jnp.transpose(out5, (2, 4, 0, 1, 3))
    return out.reshape(batch, hist, dim)


# speedup vs baseline: 1.0193x; 1.0193x over previous
import functools

import jax
import jax.numpy as jnp
from jax import lax
from jax.experimental import pallas as pl
from jax.experimental.pallas import tpu as pltpu
from jax.experimental.pallas import tpu_sc as plsc

L = 16


@functools.cache
def _make_gather(B: int, H: int, V: int, D: int):
    info = plsc.get_sparse_core_info()
    nc, ns = info.num_cores, info.num_subcores
    nw = nc * ns
    nb = B // nw
    nbt = nb // 128
    dt, di = D // 8, 8
    assert nb * nw == B and nbt * 128 == nb
    mesh = plsc.VectorSubcoreMesh(core_axis_name="c", subcore_axis_name="s")

    @functools.partial(
        pl.kernel,
        mesh=mesh,
        out_type=jax.ShapeDtypeStruct((H, dt, B // 128, di, 128), jnp.float32),
        compiler_params=pltpu.CompilerParams(
            use_tc_tiling_on_sc=False, needs_layout_passes=False,
            disable_bounds_checks=True),
        scratch_types=[
            pltpu.VMEM((V * D * 8,), jnp.float32),
            pltpu.VMEM((nb * H,), jnp.int32),
            pltpu.VMEM((2, dt, nbt, di, 128), jnp.float32),
            pltpu.SemaphoreType.DMA((2,)),
        ],
    )
    def gather_kernel(tab8_hbm, idx_hbm, out_hbm, tab8_v, idx_v, stg_v, ssem):
        cid = lax.axis_index("c")
        sid = lax.axis_index("s")
        wid = sid * nc + cid

        pltpu.sync_copy(tab8_hbm, tab8_v)
        pltpu.sync_copy(idx_hbm.at[pl.ds(wid * nb * H, nb * H)], idx_v)

        iota = lax.iota(jnp.int32, L)
        iota8 = lax.rem(iota, jnp.full((L,), 8, jnp.int32))
        iota_h = iota * H

        def store_dst(h):
            return out_hbm.at[h, :, pl.ds(wid * nbt, nbt)]

        def start_store(p, h):
            pltpu.async_copy(stg_v.at[p], store_dst(h), ssem.at[p])

        def wait_store(p, h):
            pltpu.make_async_copy(stg_v.at[p], store_dst(h),
                                  ssem.at[p]).wait()

        def produce(p, h):
            @plsc.parallel_loop(0, nbt * (128 // L), 1, unroll=4)
            def _(c):
                btl = lax.shift_right_logical(c, 3)
                g = lax.bitwise_and(c, 7)
                t0 = btl * 128 + g * L
                idxvals = plsc.load_gather(idx_v, [iota_h + (t0 * H + h)])
                base = idxvals * (D * 8) + iota8
                for d in range(D):
                    vals = plsc.load_gather(tab8_v, [base + d * 8])
                    stg_v[p, d // di, btl, d % di, pl.ds(g * L, L)] = vals

        def block(j, carry):
            for p in range(2):
                h = 2 * j + p

                @pl.when(j > 0)
                def _():
                    wait_store(p, h - 2)

                produce(p, h)
                start_store(p, h)
            return carry

        lax.fori_loop(0, H // 2, block, 0)
        wait_store(0, H - 2)
        wait_store(1, H - 1)

    return gather_kernel


def kernel(inputs, table):
    batch, hist = inputs.shape
    rows, dim = table.shape
    flat_idx = inputs.reshape(batch * hist)
    table_8 = jnp.tile(table.reshape(rows * dim, 1), (1, 8)).reshape(-1)
    out5 = _make_gather(batch, hist, rows, dim)(table_8, flat_idx)
    out = jnp.transpose(out5, (2, 4, 0, 1, 3))
    return out.reshape(batch, hist, dim)
